# baseline (speedup 1.0000x reference)
import functools

import jax
import jax.numpy as jnp
from jax import lax
from jax.experimental import pallas as pl
from jax.experimental.pallas import tpu as pltpu
from jax.experimental.pallas import tpu_sc as plsc


def _argmax_body(state_ref, mem_ref, idx_ref):
    sims = lax.dot_general(
        state_ref[...], mem_ref[...],
        dimension_numbers=(((1,), (1,)), ((), ())),
        preferred_element_type=jnp.float32,
        precision=lax.Precision.HIGHEST,
    )
    heads = sims.shape[1]
    m = jnp.max(sims, axis=1, keepdims=True)
    rev = float(heads) - lax.broadcasted_iota(jnp.int32, sims.shape, 1).astype(jnp.float32)
    masked = jnp.where(sims == m, rev, 0.0)
    win = jnp.max(masked, axis=1)
    idx_ref[pl.program_id(0), :] = (float(heads) - win).astype(jnp.int32)


def _compute_indices(state, memories, blk):
    b, d = state.shape
    heads = memories.shape[0]
    n_blk = b // blk
    return pl.pallas_call(
        _argmax_body,
        grid=(n_blk,),
        in_specs=[
            pl.BlockSpec((blk, d), lambda i: (i, 0)),
            pl.BlockSpec((heads, d), lambda i: (0, 0)),
        ],
        out_specs=pl.BlockSpec((n_blk, blk), lambda i: (0, 0)),
        out_shape=jax.ShapeDtypeStruct((n_blk, blk), jnp.int32),
    )(state, memories)


def _make_sc_gather(b, blk, act_dim):
    info = plsc.get_sparse_core_info()
    nc, ns = info.num_cores, info.num_subcores
    nw = nc * ns
    b_per_w = b // nw
    mesh = plsc.VectorSubcoreMesh(core_axis_name="c", subcore_axis_name="s")

    @functools.partial(
        pl.kernel,
        mesh=mesh,
        out_type=jax.ShapeDtypeStruct((b, act_dim), jnp.float32),
        scratch_types=[
            pltpu.VMEM((b_per_w,), jnp.int32),
            pltpu.VMEM((b_per_w, act_dim), jnp.float32),
            pltpu.SemaphoreType.DMA,
        ],
    )
    def gather(table_hbm, idx_hbm, out_hbm, idx_v, rows_v, sem):
        wid = lax.axis_index("s") * nc + lax.axis_index("c")
        base = wid * b_per_w
        row = base // blk
        col = base - row * blk
        pltpu.sync_copy(idx_hbm.at[row, pl.ds(col, b_per_w)], idx_v)
        pltpu.async_copy(table_hbm.at[idx_v], rows_v, sem).wait()
        pltpu.sync_copy(rows_v, out_hbm.at[pl.ds(base, b_per_w)])

    return gather


@jax.jit
def kernel(state, memories, logits):
    b = state.shape[0]
    act_dim = logits.shape[1]
    half = b // 2
    gather = _make_sc_gather(half, half, act_dim)
    outs = []
    for h in range(2):
        sh = lax.slice_in_dim(state, h * half, (h + 1) * half, axis=0)
        idx2d = _compute_indices(sh, memories, half)
        outs.append(gather(logits, idx2d))
    return jnp.concatenate(outs, axis=0)


# speedup vs baseline: 1.2147x; 1.2147x over previous
import functools

import jax
import jax.numpy as jnp
from jax import lax
from jax.experimental import pallas as pl
from jax.experimental.pallas import tpu as pltpu
from jax.experimental.pallas import tpu_sc as plsc


def _argmax_body(state_ref, mem_ref, idx_ref):
    sims = lax.dot_general(
        state_ref[...], mem_ref[...],
        dimension_numbers=(((1,), (1,)), ((), ())),
        preferred_element_type=jnp.float32,
        precision=lax.Precision.HIGHEST,
    )
    heads = sims.shape[1]
    m = jnp.max(sims, axis=1, keepdims=True)
    rev = float(heads) - lax.broadcasted_iota(jnp.int32, sims.shape, 1).astype(jnp.float32)
    masked = jnp.where(sims == m, rev, 0.0)
    win = jnp.max(masked, axis=1)
    idx_ref[pl.program_id(0), :] = (float(heads) - win).astype(jnp.int32)


def _compute_indices(state, memories, blk):
    b, d = state.shape
    heads = memories.shape[0]
    n_blk = b // blk
    return pl.pallas_call(
        _argmax_body,
        grid=(n_blk,),
        in_specs=[
            pl.BlockSpec((blk, d), lambda i: (i, 0)),
            pl.BlockSpec((heads, d), lambda i: (0, 0)),
        ],
        out_specs=pl.BlockSpec((n_blk, blk), lambda i: (0, 0)),
        out_shape=jax.ShapeDtypeStruct((n_blk, blk), jnp.int32),
    )(state, memories)


def _make_sc_gather(b, blk, act_dim):
    info = plsc.get_sparse_core_info()
    nc, ns = info.num_cores, info.num_subcores
    nw = nc * ns
    b_per_w = b // nw
    mesh = plsc.VectorSubcoreMesh(core_axis_name="c", subcore_axis_name="s")

    hw = b_per_w // 2

    @functools.partial(
        pl.kernel,
        mesh=mesh,
        out_type=jax.ShapeDtypeStruct((b, act_dim), jnp.float32),
        scratch_types=[
            pltpu.VMEM((b_per_w,), jnp.int32),
            pltpu.VMEM((b_per_w, act_dim), jnp.float32),
            pltpu.SemaphoreType.DMA,
            pltpu.SemaphoreType.DMA,
            pltpu.SemaphoreType.DMA,
        ],
    )
    def gather(table_hbm, idx_hbm, out_hbm, idx_v, rows_v, sem0, sem1, semw):
        wid = lax.axis_index("s") * nc + lax.axis_index("c")
        base = wid * b_per_w
        row = base // blk
        col = base - row * blk
        pltpu.sync_copy(idx_hbm.at[row, pl.ds(col, b_per_w)], idx_v)
        g0 = pltpu.async_copy(
            table_hbm.at[idx_v.at[pl.ds(0, hw)]], rows_v.at[pl.ds(0, hw)], sem0)
        g1 = pltpu.async_copy(
            table_hbm.at[idx_v.at[pl.ds(hw, hw)]], rows_v.at[pl.ds(hw, hw)], sem1)
        g0.wait()
        w0 = pltpu.async_copy(
            rows_v.at[pl.ds(0, hw)], out_hbm.at[pl.ds(base, hw)], semw)
        g1.wait()
        pltpu.sync_copy(rows_v.at[pl.ds(hw, hw)], out_hbm.at[pl.ds(base + hw, hw)])
        w0.wait()

    return gather


@jax.jit
def kernel(state, memories, logits):
    b = state.shape[0]
    act_dim = logits.shape[1]
    blk = 2048
    idx2d = _compute_indices(state, memories, blk)
    out = _make_sc_gather(b, blk, act_dim)(logits, idx2d)
    return out


# speedup vs baseline: 1.3254x; 1.0911x over previous
import functools

import jax
import jax.numpy as jnp
from jax import lax
from jax.experimental import pallas as pl
from jax.experimental.pallas import tpu as pltpu
from jax.experimental.pallas import tpu_sc as plsc


def _argmax_body(state_ref, mem_ref, idx_ref):
    sims = lax.dot_general(
        mem_ref[...], state_ref[...],
        dimension_numbers=(((1,), (1,)), ((), ())),
        preferred_element_type=jnp.float32,
        precision=lax.Precision.HIGHEST,
    )
    heads = sims.shape[0]
    m = jnp.max(sims, axis=0, keepdims=True)
    rev = float(heads) - lax.broadcasted_iota(jnp.int32, sims.shape, 0).astype(jnp.float32)
    masked = jnp.where(sims == m, rev, 0.0)
    win = jnp.max(masked, axis=0)
    idx_ref[pl.program_id(0), :] = (float(heads) - win).astype(jnp.int32)


def _compute_indices(state, memories, blk):
    b, d = state.shape
    heads = memories.shape[0]
    n_blk = b // blk
    return pl.pallas_call(
        _argmax_body,
        grid=(n_blk,),
        in_specs=[
            pl.BlockSpec((blk, d), lambda i: (i, 0)),
            pl.BlockSpec((heads, d), lambda i: (0, 0)),
        ],
        out_specs=pl.BlockSpec((n_blk, blk), lambda i: (0, 0)),
        out_shape=jax.ShapeDtypeStruct((n_blk, blk), jnp.int32),
    )(state, memories)


def _make_sc_gather(b, blk, act_dim):
    info = plsc.get_sparse_core_info()
    nc, ns = info.num_cores, info.num_subcores
    nw = nc * ns
    b_per_w = b // nw
    mesh = plsc.VectorSubcoreMesh(core_axis_name="c", subcore_axis_name="s")

    hw = b_per_w // 2

    @functools.partial(
        pl.kernel,
        mesh=mesh,
        out_type=jax.ShapeDtypeStruct((b, act_dim), jnp.float32),
        scratch_types=[
            pltpu.VMEM((b_per_w,), jnp.int32),
            pltpu.VMEM((b_per_w, act_dim), jnp.float32),
            pltpu.SemaphoreType.DMA,
            pltpu.SemaphoreType.DMA,
            pltpu.SemaphoreType.DMA,
        ],
    )
    def gather(table_hbm, idx_hbm, out_hbm, idx_v, rows_v, sem0, sem1, semw):
        wid = lax.axis_index("s") * nc + lax.axis_index("c")
        base = wid * b_per_w
        row = base // blk
        col = base - row * blk
        pltpu.sync_copy(idx_hbm.at[row, pl.ds(col, b_per_w)], idx_v)
        g0 = pltpu.async_copy(
            table_hbm.at[idx_v.at[pl.ds(0, hw)]], rows_v.at[pl.ds(0, hw)], sem0)
        g1 = pltpu.async_copy(
            table_hbm.at[idx_v.at[pl.ds(hw, hw)]], rows_v.at[pl.ds(hw, hw)], sem1)
        g0.wait()
        w0 = pltpu.async_copy(
            rows_v.at[pl.ds(0, hw)], out_hbm.at[pl.ds(base, hw)], semw)
        g1.wait()
        pltpu.sync_copy(rows_v.at[pl.ds(hw, hw)], out_hbm.at[pl.ds(base + hw, hw)])
        w0.wait()

    return gather


@jax.jit
def kernel(state, memories, logits):
    b = state.shape[0]
    act_dim = logits.shape[1]
    blk = 2048
    idx2d = _compute_indices(state, memories, blk)
    out = _make_sc_gather(b, blk, act_dim)(logits, idx2d)
    return out
